# Initial kernel scaffold; baseline (speedup 1.0000x reference)
import jax
import jax.numpy as jnp
from jax.experimental import pallas as pl


def kernel(values, offsets, table):
    raise NotImplementedError("write your pallas kernel here")


# speedup vs baseline: 2.6399x; 2.6399x over previous
import jax
import jax.numpy as jnp
from jax import lax
from jax.experimental import pallas as pl
from jax.experimental.pallas import tpu as pltpu
from jax.experimental.pallas import tpu_sc as plsc

F = 26
B = 4096
D = 32
NB = F * B
TI = NB * 2
NBH = NB // 2
NS = 16
VPT = TI // NS
OPT = NB // NS
BPT = NBH // NS
CH = 256
NCH = VPT // CH
WCH = 256
NWCH = BPT // WCH
W = TI // 2
WPT = W // NS


def _cumsum16(v, iota):
    for k in (1, 2, 4, 8):
        v = v + jnp.where(iota >= k,
                          jnp.take(v, jnp.maximum(iota - k, 0), mode="wrap"), 0)
    return v


def _body(values, offsets, table, zrows, zwords, ones_h, out, seg,
          acc, dwin, tsums, vbuf, vals2d, idx2d, segb, wbuf, tbuf, tv16,
          ones_v, sem):
    c = lax.axis_index("c")
    s = lax.axis_index("s")
    iota = lax.iota(jnp.int32, 16)

    def zero_dwin():
        pltpu.sync_copy(zwords, dwin.at[pl.ds(s * WPT, WPT)])
        @pl.when(s == 0)
        def _():
            pltpu.sync_copy(zwords.at[pl.ds(0, 16)], dwin.at[pl.ds(W, 16)])

    pltpu.sync_copy(ones_h, ones_v)
    for i in range(3):
        pltpu.sync_copy(zrows, acc.at[pl.ds(s * BPT + i * 1024, 1024)])
    pltpu.sync_copy(zrows.at[pl.ds(0, 256)], acc.at[pl.ds(s * BPT + 3072, 256)])
    zero_dwin()
    plsc.subcore_barrier()

    for r in (0, 1):
        def off_it(i, t):
            pos = s * OPT + i * 256
            for j in range(2):
                pltpu.sync_copy(offsets.at[pl.ds(pos + j * 128, 128)],
                                idx2d.at[j])
            def tr_it(k, t2):
                v = idx2d[k >> 3, pl.ds((k & 7) * 16, 16)]
                d = v - r * W
                ok = (d >= 0) & (d < W)
                idx2d[k >> 3, pl.ds((k & 7) * 16, 16)] = jnp.where(ok, d, W)
                return t2
            lax.fori_loop(0, 16, tr_it, 0)
            for j in range(2):
                pltpu.sync_copy(ones_v, dwin.at[idx2d.at[j]], add=True)
            return t
        lax.fori_loop(0, OPT // 256, off_it, 0)
        plsc.subcore_barrier()

        @pl.when((s >= 8 * r) & (s < 8 * (r + 1)))
        def _():
            base = (s - 8 * r) * VPT
            def seg_it(q, carry_v):
                pltpu.sync_copy(dwin.at[pl.ds(base + q * 256, 256)], wbuf)
                def cs_it(g, cv):
                    v = _cumsum16(wbuf[pl.ds(g * 16, 16)], iota) + cv
                    wbuf[pl.ds(g * 16, 16)] = v
                    return jnp.take(v, iota * 0 + 15, mode="wrap")
                carry_v = lax.fori_loop(0, 16, cs_it, carry_v)
                pltpu.sync_copy(wbuf, seg.at[c, pl.ds(s * VPT + q * 256, 256)])
                return carry_v
            carry_v = lax.fori_loop(0, VPT // 256, seg_it,
                                    jnp.zeros((16,), jnp.int32))
            tv16[...] = carry_v
            pltpu.sync_copy(tv16, tsums.at[s])
        plsc.subcore_barrier()

        if r == 0:
            zero_dwin()
            plsc.subcore_barrier()

    pltpu.sync_copy(tsums, tbuf)
    totv = jnp.zeros((16,), jnp.int32)
    for l in range(16):
        totv = jnp.where(iota == l, tbuf[l, pl.ds(0, 16)], totv)
    mbv = jnp.where(iota < s, totv, 0)
    for k in (1, 2, 4, 8):
        mbv = mbv + jnp.take(mbv, iota ^ k, mode="wrap")
    adjv = mbv + (-1 - c * NBH)

    def main_it(i, t):
        pos = s * VPT + i * CH
        for j in range(2):
            pltpu.sync_copy(values.at[pl.ds(pos + j * 128, 128)], vals2d.at[j])
        pltpu.sync_copy(seg.at[c, pl.ds(pos, CH)], segb)

        def sidx_it(k, t2):
            d = segb[pl.ds(k * 16, 16)] + adjv
            ok = (d >= 0) & (d < NBH)
            idx2d[k >> 3, pl.ds((k & 7) * 16, 16)] = jnp.where(ok, d, NBH)
            return t2
        lax.fori_loop(0, CH // 16, sidx_it, 0)

        descs = [pltpu.async_copy(table.at[vals2d.at[j]],
                                  vbuf.at[pl.ds(j * 128, 128)], sem)
                 for j in range(2)]
        for j in range(2):
            descs[j].wait()
            pltpu.sync_copy(vbuf.at[pl.ds(j * 128, 128)],
                            acc.at[idx2d.at[j]], add=True)
        return t
    lax.fori_loop(0, NCH, main_it, 0)
    plsc.subcore_barrier()

    def wr_it(i, t):
        r0 = s * BPT + i * WCH
        pltpu.sync_copy(acc.at[pl.ds(r0, WCH)], vbuf)

        def didx_it(k, t2):
            gv = (c * NBH + r0 + k * 16) + iota
            fv = gv >> 12
            bv = gv & (B - 1)
            idx2d[k >> 3, pl.ds((k & 7) * 16, 16)] = bv * F + fv
            return t2
        lax.fori_loop(0, WCH // 16, didx_it, 0)
        for j in range(WCH // 128):
            pltpu.sync_copy(vbuf.at[pl.ds(j * 128, 128)], out.at[idx2d.at[j]])
        return t
    lax.fori_loop(0, NWCH, wr_it, 0)


_fused = pl.kernel(
    _body,
    out_type=(jax.ShapeDtypeStruct((NB, D), jnp.float32),
              jax.ShapeDtypeStruct((2, TI), jnp.int32)),
    mesh=plsc.VectorSubcoreMesh(core_axis_name="c", subcore_axis_name="s"),
    compiler_params=pltpu.CompilerParams(use_tc_tiling_on_sc=False),
    scratch_types=[
        pltpu.VMEM_SHARED((NBH + 16, D), jnp.float32),
        pltpu.VMEM_SHARED((W + 16,), jnp.int32),
        pltpu.VMEM_SHARED((16, 16), jnp.int32),
        pltpu.VMEM((CH, D), jnp.float32),
        pltpu.VMEM((2, 128), jnp.int32),
        pltpu.VMEM((2, 128), jnp.int32),
        pltpu.VMEM((CH,), jnp.int32),
        pltpu.VMEM((256,), jnp.int32),
        pltpu.VMEM((16, 16), jnp.int32),
        pltpu.VMEM((16,), jnp.int32),
        pltpu.VMEM((128,), jnp.int32),
        pltpu.SemaphoreType.DMA,
    ],
)


@jax.jit
def kernel(values, offsets, table):
    zrows = jnp.zeros((1024, D), jnp.float32)
    zwords = jnp.zeros((WPT,), jnp.int32)
    ones_h = jnp.ones((128,), jnp.int32)
    out2d, _ = _fused(values, offsets, table, zrows, zwords, ones_h)
    return out2d.reshape(B, F, D)


# speedup vs baseline: 2.6953x; 1.0210x over previous
import jax
import jax.numpy as jnp
from jax import lax
from jax.experimental import pallas as pl
from jax.experimental.pallas import tpu as pltpu
from jax.experimental.pallas import tpu_sc as plsc

F = 26
B = 4096
D = 32
NB = F * B
TI = NB * 2
NBH = NB // 2
NS = 16
VPT = TI // NS
OPT = NB // NS
BPT = NBH // NS
SC_ = 1024
NSC = VPT // SC_
NG = SC_ // 128
WCH = 256
NWCH = BPT // WCH
W = TI // 2
WPT = W // NS


def _cumsum16(v, iota):
    for k in (1, 2, 4, 8):
        v = v + jnp.where(iota >= k,
                          jnp.take(v, jnp.maximum(iota - k, 0), mode="wrap"), 0)
    return v


def _body(values, offsets, table, zrows, zwords, ones_h, out, seg,
          acc, dwin, tsums, vbuf, vals1, segb, idx2d, wbuf, tbuf, tv16,
          ones_v, semv, semg0, semg1, sems0, sems1):
    c = lax.axis_index("c")
    s = lax.axis_index("s")
    iota = lax.iota(jnp.int32, 16)
    semg = (semg0, semg1)
    sems = (sems0, sems1)

    def zero_dwin():
        pltpu.sync_copy(zwords, dwin.at[pl.ds(s * WPT, WPT)])
        @pl.when(s == 0)
        def _():
            pltpu.sync_copy(zwords.at[pl.ds(0, 16)], dwin.at[pl.ds(W, 16)])

    pltpu.sync_copy(ones_h, ones_v)
    for i in range(3):
        pltpu.sync_copy(zrows, acc.at[pl.ds(s * BPT + i * 1024, 1024)])
    pltpu.sync_copy(zrows.at[pl.ds(0, 256)], acc.at[pl.ds(s * BPT + 3072, 256)])
    zero_dwin()
    plsc.subcore_barrier()

    for r in (0, 1):
        def off_it(i, t):
            pos = s * OPT + i * 256
            for j in range(2):
                pltpu.sync_copy(offsets.at[pl.ds(pos + j * 128, 128)],
                                idx2d.at[j])
            def tr_it(k, t2):
                v = idx2d[k >> 3, pl.ds((k & 7) * 16, 16)]
                d = v - r * W
                ok = (d >= 0) & (d < W)
                idx2d[k >> 3, pl.ds((k & 7) * 16, 16)] = jnp.where(ok, d, W)
                return t2
            lax.fori_loop(0, 16, tr_it, 0)
            for j in range(2):
                pltpu.sync_copy(ones_v, dwin.at[idx2d.at[j]], add=True)
            return t
        lax.fori_loop(0, OPT // 256, off_it, 0)
        plsc.subcore_barrier()

        @pl.when((s >= 8 * r) & (s < 8 * (r + 1)))
        def _():
            base = (s - 8 * r) * VPT
            def seg_it(q, carry_v):
                pltpu.sync_copy(dwin.at[pl.ds(base + q * SC_, SC_)], wbuf)
                def cs_it(g, cv):
                    v = _cumsum16(wbuf[pl.ds(g * 16, 16)], iota) + cv
                    wbuf[pl.ds(g * 16, 16)] = v
                    return jnp.take(v, iota * 0 + 15, mode="wrap")
                carry_v = lax.fori_loop(0, SC_ // 16, cs_it, carry_v)
                pltpu.sync_copy(wbuf, seg.at[c, pl.ds(s * VPT + q * SC_, SC_)])
                return carry_v
            carry_v = lax.fori_loop(0, NSC, seg_it, jnp.zeros((16,), jnp.int32))
            tv16[...] = carry_v
            pltpu.sync_copy(tv16, tsums.at[s])
        plsc.subcore_barrier()

        if r == 0:
            zero_dwin()
            plsc.subcore_barrier()

    pltpu.sync_copy(tsums, tbuf)
    totv = jnp.zeros((16,), jnp.int32)
    for l in range(16):
        totv = jnp.where(iota == l, tbuf[l, pl.ds(0, 16)], totv)
    mbv = jnp.where(iota < s, totv, 0)
    for k in (1, 2, 4, 8):
        mbv = mbv + jnp.take(mbv, iota ^ k, mode="wrap")
    adjv = mbv + (-1 - c * NBH)

    def fire_fetch(i, p):
        pos = s * VPT + i * SC_
        pltpu.async_copy(values.at[pl.ds(pos, SC_)],
                         vals1.at[pl.ds(p * SC_, SC_)], semv)
        pltpu.async_copy(seg.at[c, pl.ds(pos, SC_)],
                         segb.at[pl.ds(p * SC_, SC_)], semv)

    def wait_fetch(p):
        for _ in range(2):
            pltpu.make_async_copy(values.at[pl.ds(0, SC_)],
                                  vals1.at[pl.ds(p * SC_, SC_)], semv).wait()

    fire_fetch(0, 0)

    def main_it(i, t):
        p = i & 1
        pb = p * SC_
        wait_fetch(p)
        @pl.when(i < NSC - 1)
        def _():
            fire_fetch(i + 1, 1 - p)

        def gath(g):
            return pltpu.async_copy(
                table.at[vals1.at[pl.ds(pb + g * 128, 128)]],
                vbuf.at[pl.ds((g & 1) * 128, 128)], semg[g & 1])

        def scat(g):
            return pltpu.async_copy(
                vbuf.at[pl.ds((g & 1) * 128, 128)],
                acc.at[idx2d.at[g]], sems[g & 1], add=True)

        dg = {0: gath(0), 1: gath(1)}
        ds_ = {}
        for g in range(NG):
            def idxc(k, t2):
                d = segb[pl.ds(pb + g * 128 + k * 16, 16)] + adjv
                ok = (d >= 0) & (d < NBH)
                idx2d[g, pl.ds(k * 16, 16)] = jnp.where(ok, d, NBH)
                return t2
            lax.fori_loop(0, 8, idxc, 0)
            dg[g].wait()
            ds_[g] = scat(g)
            if g + 2 < NG:
                ds_[g].wait()
                dg[g + 2] = gath(g + 2)
        ds_[NG - 2].wait()
        ds_[NG - 1].wait()
        return t
    lax.fori_loop(0, NSC, main_it, 0)
    plsc.subcore_barrier()

    def wr_it(i, t):
        r0 = s * BPT + i * WCH
        pltpu.sync_copy(acc.at[pl.ds(r0, WCH)], vbuf.at[pl.ds(0, WCH)])

        def didx_it(k, t2):
            gv = (c * NBH + r0 + k * 16) + iota
            fv = gv >> 12
            bv = gv & (B - 1)
            idx2d[k >> 3, pl.ds((k & 7) * 16, 16)] = bv * F + fv
            return t2
        lax.fori_loop(0, WCH // 16, didx_it, 0)
        for j in range(WCH // 128):
            pltpu.sync_copy(vbuf.at[pl.ds(j * 128, 128)], out.at[idx2d.at[j]])
        return t
    lax.fori_loop(0, NWCH, wr_it, 0)


_fused = pl.kernel(
    _body,
    out_type=(jax.ShapeDtypeStruct((NB, D), jnp.float32),
              jax.ShapeDtypeStruct((2, TI), jnp.int32)),
    mesh=plsc.VectorSubcoreMesh(core_axis_name="c", subcore_axis_name="s"),
    compiler_params=pltpu.CompilerParams(use_tc_tiling_on_sc=False),
    scratch_types=[
        pltpu.VMEM_SHARED((NBH + 16, D), jnp.float32),
        pltpu.VMEM_SHARED((W + 16,), jnp.int32),
        pltpu.VMEM_SHARED((16, 16), jnp.int32),
        pltpu.VMEM((256, D), jnp.float32),
        pltpu.VMEM((2 * SC_,), jnp.int32),
        pltpu.VMEM((2 * SC_,), jnp.int32),
        pltpu.VMEM((NG, 128), jnp.int32),
        pltpu.VMEM((SC_,), jnp.int32),
        pltpu.VMEM((16, 16), jnp.int32),
        pltpu.VMEM((16,), jnp.int32),
        pltpu.VMEM((128,), jnp.int32),
        pltpu.SemaphoreType.DMA,
        pltpu.SemaphoreType.DMA,
        pltpu.SemaphoreType.DMA,
        pltpu.SemaphoreType.DMA,
        pltpu.SemaphoreType.DMA,
    ],
)


@jax.jit
def kernel(values, offsets, table):
    zrows = jnp.zeros((1024, D), jnp.float32)
    zwords = jnp.zeros((WPT,), jnp.int32)
    ones_h = jnp.ones((128,), jnp.int32)
    out2d, _ = _fused(values, offsets, table, zrows, zwords, ones_h)
    return out2d.reshape(B, F, D)


# speedup vs baseline: 2.7101x; 1.0055x over previous
import jax
import jax.numpy as jnp
from jax import lax
from jax.experimental import pallas as pl
from jax.experimental.pallas import tpu as pltpu
from jax.experimental.pallas import tpu_sc as plsc

F = 26
B = 4096
D = 32
NB = F * B
TI = NB * 2
NBH = NB // 2
NS = 16
VPT = TI // NS
OPT = NB // NS
BPT = NBH // NS
SC_ = 1024
NSC = VPT // SC_
NG = SC_ // 128
WCH = 128
NWCH = BPT // WCH
W = TI // 2
WPT = W // NS


def _cumsum16(v, iota):
    for k in (1, 2, 4, 8):
        v = v + jnp.where(iota >= k,
                          jnp.take(v, jnp.maximum(iota - k, 0), mode="wrap"), 0)
    return v


def _body(values, offsets, table, zrows, zwords, ones_h, out, seg,
          acc, dwin, tsums, vbuf, vals1, segb, idx2d, wbuf, tbuf, tv16,
          ones_v, semv, semg0, semg1, sems0, sems1):
    c = lax.axis_index("c")
    s = lax.axis_index("s")
    iota = lax.iota(jnp.int32, 16)
    semg = (semg0, semg1)
    sems = (sems0, sems1)

    def zero_dwin():
        pltpu.sync_copy(zwords, dwin.at[pl.ds(s * WPT, WPT)])
        @pl.when(s == 0)
        def _():
            pltpu.sync_copy(zwords.at[pl.ds(0, 16)], dwin.at[pl.ds(W, 16)])

    _sc_init = jax.named_scope("ph_init"); _sc_init.__enter__()
    pltpu.sync_copy(ones_h, ones_v)
    for i in range(3):
        pltpu.sync_copy(zrows, acc.at[pl.ds(s * BPT + i * 1024, 1024)])
    pltpu.sync_copy(zrows.at[pl.ds(0, 256)], acc.at[pl.ds(s * BPT + 3072, 256)])
    zero_dwin()
    plsc.subcore_barrier()
    _sc_init.__exit__(None, None, None)

    for r in (0, 1):
        _sc_h = jax.named_scope("ph_hist%d" % r); _sc_h.__enter__()
        NOI = OPT // 256

        def h_load(i, p):
            pos = s * OPT + i * 256
            for j in range(2):
                pltpu.async_copy(offsets.at[pl.ds(pos + j * 128, 128)],
                                 idx2d.at[2 * p + j], semv)

        def h_wload(p):
            for j in range(2):
                pltpu.make_async_copy(offsets.at[pl.ds(0, 128)],
                                      idx2d.at[2 * p + j], semv).wait()

        def h_wscat(p):
            for j in range(2):
                pltpu.make_async_copy(ones_v, dwin.at[pl.ds(0, 128)],
                                      sems0).wait()

        h_load(0, 0)

        def off_it(i, t):
            p = i & 1
            @pl.when(i > 0)
            def _():
                h_wscat(1 - p)
            h_wload(p)
            @pl.when(i < NOI - 1)
            def _():
                h_load(i + 1, 1 - p)
            def tr_it(k, t2):
                row = 2 * p + (k >> 3)
                v = idx2d[row, pl.ds((k & 7) * 16, 16)]
                d = v - r * W
                idx2d[row, pl.ds((k & 7) * 16, 16)] = jnp.where(
                    d < 0, W, jnp.minimum(d, W))
                return t2
            lax.fori_loop(0, 16, tr_it, 0)
            for j in range(2):
                pltpu.async_copy(ones_v, dwin.at[idx2d.at[2 * p + j]],
                                 sems0, add=True)
            return t
        lax.fori_loop(0, NOI, off_it, 0)
        h_wscat((NOI - 1) & 1)
        plsc.subcore_barrier()
        _sc_h.__exit__(None, None, None)
        _sc_cs = jax.named_scope("ph_cumsum%d" % r); _sc_cs.__enter__()

        @pl.when((s >= 8 * r) & (s < 8 * (r + 1)))
        def _():
            base = (s - 8 * r) * VPT
            def seg_it(q, carry_v):
                pltpu.sync_copy(dwin.at[pl.ds(base + q * SC_, SC_)], wbuf)
                def cs_it(g, cv):
                    v = _cumsum16(wbuf[pl.ds(g * 16, 16)], iota) + cv
                    wbuf[pl.ds(g * 16, 16)] = v
                    return jnp.take(v, iota * 0 + 15, mode="wrap")
                carry_v = lax.fori_loop(0, SC_ // 16, cs_it, carry_v)
                pltpu.sync_copy(wbuf, seg.at[c, pl.ds(s * VPT + q * SC_, SC_)])
                return carry_v
            carry_v = lax.fori_loop(0, NSC, seg_it, jnp.zeros((16,), jnp.int32))
            tv16[...] = carry_v
            pltpu.sync_copy(tv16, tsums.at[s])
        plsc.subcore_barrier()
        _sc_cs.__exit__(None, None, None)

        if r == 0:
            zero_dwin()
            plsc.subcore_barrier()

    pltpu.sync_copy(tsums, tbuf)
    totv = jnp.zeros((16,), jnp.int32)
    for l in range(16):
        totv = jnp.where(iota == l, tbuf[l, pl.ds(0, 16)], totv)
    mbv = jnp.where(iota < s, totv, 0)
    for k in (1, 2, 4, 8):
        mbv = mbv + jnp.take(mbv, iota ^ k, mode="wrap")
    adjv = mbv + (-1 - c * NBH)

    def fire_fetch(i, p):
        pos = s * VPT + i * SC_
        pltpu.async_copy(values.at[pl.ds(pos, SC_)],
                         vals1.at[pl.ds(p * SC_, SC_)], semv)
        pltpu.async_copy(seg.at[c, pl.ds(pos, SC_)],
                         segb.at[pl.ds(p * SC_, SC_)], semv)

    def wait_fetch(p):
        for _ in range(2):
            pltpu.make_async_copy(values.at[pl.ds(0, SC_)],
                                  vals1.at[pl.ds(p * SC_, SC_)], semv).wait()

    _sc_m = jax.named_scope("ph_main"); _sc_m.__enter__()
    fire_fetch(0, 0)

    def main_it(i, t):
        p = i & 1
        pb = p * SC_
        wait_fetch(p)
        @pl.when(i < NSC - 1)
        def _():
            fire_fetch(i + 1, 1 - p)

        def gath(g):
            return pltpu.async_copy(
                table.at[vals1.at[pl.ds(pb + g * 128, 128)]],
                vbuf.at[pl.ds((g & 1) * 128, 128)], semg[g & 1])

        def scat(g):
            return pltpu.async_copy(
                vbuf.at[pl.ds((g & 1) * 128, 128)],
                acc.at[idx2d.at[g]], sems[g & 1], add=True)

        dg = {0: gath(0), 1: gath(1)}
        ds_ = {}
        for g in range(NG):
            def idxc(k, t2):
                d = segb[pl.ds(pb + g * 128 + k * 16, 16)] + adjv
                ok = (d >= 0) & (d < NBH)
                idx2d[g, pl.ds(k * 16, 16)] = jnp.where(ok, d, NBH)
                return t2
            lax.fori_loop(0, 8, idxc, 0)
            dg[g].wait()
            ds_[g] = scat(g)
            if g + 2 < NG:
                ds_[g].wait()
                dg[g + 2] = gath(g + 2)
        ds_[NG - 2].wait()
        ds_[NG - 1].wait()
        return t
    lax.fori_loop(0, NSC, main_it, 0)
    plsc.subcore_barrier()
    _sc_m.__exit__(None, None, None)
    _sc_w = jax.named_scope("ph_writeout"); _sc_w.__enter__()

    def w_read(i, p):
        pltpu.async_copy(acc.at[pl.ds(s * BPT + i * 128, 128)],
                         vbuf.at[pl.ds(p * 128, 128)], semg[p])

    def w_wread(p):
        pltpu.make_async_copy(acc.at[pl.ds(0, 128)],
                              vbuf.at[pl.ds(p * 128, 128)], semg[p]).wait()

    def w_wscat(p):
        pltpu.make_async_copy(vbuf.at[pl.ds(p * 128, 128)],
                              out.at[pl.ds(0, 128)], sems[p]).wait()

    w_read(0, 0)
    w_read(1, 1)

    def wr_it(ii, t):
        for p in (0, 1):
            g = 2 * ii + p

            def didx_it(k, t2):
                gv = (c * NBH + s * BPT + g * 128 + k * 16) + iota
                fv = gv >> 12
                bv = gv & (B - 1)
                idx2d[p, pl.ds(k * 16, 16)] = bv * F + fv
                return t2
            lax.fori_loop(0, 8, didx_it, 0)
            w_wread(p)
            pltpu.async_copy(vbuf.at[pl.ds(p * 128, 128)],
                             out.at[idx2d.at[p]], sems[p])
            @pl.when(g < NWCH - 2)
            def _():
                w_wscat(p)
                w_read(g + 2, p)
        return t
    lax.fori_loop(0, NWCH // 2, wr_it, 0)
    w_wscat(0)
    w_wscat(1)
    _sc_w.__exit__(None, None, None)


_fused = pl.kernel(
    _body,
    out_type=(jax.ShapeDtypeStruct((NB, D), jnp.float32),
              jax.ShapeDtypeStruct((2, TI), jnp.int32)),
    mesh=plsc.VectorSubcoreMesh(core_axis_name="c", subcore_axis_name="s"),
    compiler_params=pltpu.CompilerParams(use_tc_tiling_on_sc=False),
    scratch_types=[
        pltpu.VMEM_SHARED((NBH + 16, D), jnp.float32),
        pltpu.VMEM_SHARED((W + 16,), jnp.int32),
        pltpu.VMEM_SHARED((16, 16), jnp.int32),
        pltpu.VMEM((256, D), jnp.float32),
        pltpu.VMEM((2 * SC_,), jnp.int32),
        pltpu.VMEM((2 * SC_,), jnp.int32),
        pltpu.VMEM((NG, 128), jnp.int32),
        pltpu.VMEM((SC_,), jnp.int32),
        pltpu.VMEM((16, 16), jnp.int32),
        pltpu.VMEM((16,), jnp.int32),
        pltpu.VMEM((128,), jnp.int32),
        pltpu.SemaphoreType.DMA,
        pltpu.SemaphoreType.DMA,
        pltpu.SemaphoreType.DMA,
        pltpu.SemaphoreType.DMA,
        pltpu.SemaphoreType.DMA,
    ],
)


@jax.jit
def kernel(values, offsets, table):
    zrows = jnp.zeros((1024, D), jnp.float32)
    zwords = jnp.zeros((WPT,), jnp.int32)
    ones_h = jnp.ones((128,), jnp.int32)
    out2d, _ = _fused(values, offsets, table, zrows, zwords, ones_h)
    return out2d.reshape(B, F, D)
